# baseline (speedup 1.0000x reference)
import functools

import jax
import jax.numpy as jnp
from jax import lax
from jax.experimental import pallas as pl
from jax.experimental.pallas import tpu as pltpu
from jax.experimental.pallas import tpu_sc as plsc

_WTAB = [0.7, 0.9, 0.7, 0.9, 0.3, 0.7, 0.3, 0.9, 0.3, 0.3, 0.9, 0.7, 0.1,
         0.9, 0.5, 0.9, 0.5, 0.5, 0.1, 0.3, 0.7, 0.9, 0.9, 0.9, 0.9, 0.9]
_EPS = 1e-5


def _fill_rows(buf, nrows, ncols, value):
    vec = jnp.full((16,), value, jnp.float32)

    def row(i, carry):
        for c in range(ncols // 16):
            buf[i, pl.ds(c * 16, 16)] = vec
        return carry

    lax.fori_loop(0, nrows, row, 0)


@functools.lru_cache(maxsize=None)
def _make_sc_agg(n_acc, erows, dh, chr_):
    rt = erows // 16
    nch = n_acc // 128
    zch = (nch + 15) // 16
    mesh = plsc.VectorSubcoreMesh(core_axis_name="c", subcore_axis_name="s")

    def body(xs0, xs1, srcp, dstp, out0, out1, sidx, didx, rows, zbuf, acc, sem):
        cid = lax.axis_index("c")
        sid = lax.axis_index("s")

        _fill_rows(zbuf, 128, dh, 0.0)

        def zacc(j, carry):
            ch = sid + 16 * j

            @pl.when(ch < nch)
            def _():
                pltpu.sync_copy(zbuf, acc.at[pl.ds(ch * 128, 128)])

            return carry

        lax.fori_loop(0, zch, zacc, 0)
        plsc.subcore_barrier()

        def run(xs, out):
            def chunk(g, carry):
                base = sid * rt + g * chr_
                pltpu.sync_copy(srcp.at[pl.ds(base, chr_)], sidx)
                pltpu.sync_copy(dstp.at[pl.ds(base, chr_)], didx)
                cps = [pltpu.async_copy(xs.at[sidx.at[j]], rows.at[j], sem)
                       for j in range(chr_)]
                for cp in cps:
                    cp.wait()
                for j in range(chr_):
                    pltpu.sync_copy(rows.at[j], acc.at[didx.at[j]], add=True)
                return carry

            lax.fori_loop(0, rt // chr_, chunk, 0)
            plsc.subcore_barrier()

            def wb(j, carry):
                ch = sid + 16 * j

                @pl.when(ch < nch)
                def _():
                    pltpu.sync_copy(acc.at[pl.ds(ch * 128, 128)],
                                    out.at[pl.ds(ch * 128, 128)])

                return carry

            lax.fori_loop(0, zch, wb, 0)

        @pl.when(cid == 0)
        def _():
            run(xs0, out0)

        @pl.when(cid == 1)
        def _():
            run(xs1, out1)

    return pl.kernel(
        body,
        mesh=mesh,
        out_type=[jax.ShapeDtypeStruct((n_acc, dh), jnp.float32),
                  jax.ShapeDtypeStruct((n_acc, dh), jnp.float32)],
        scratch_types=[
            pltpu.VMEM((chr_, 128), jnp.int32),
            pltpu.VMEM((chr_, 128), jnp.int32),
            pltpu.VMEM((chr_, 128, dh), jnp.float32),
            pltpu.VMEM((128, dh), jnp.float32),
            pltpu.VMEM_SHARED((n_acc, dh), jnp.float32),
            pltpu.SemaphoreType.DMA,
        ],
        compiler_params=pltpu.CompilerParams(use_tc_tiling_on_sc=False),
    )


@functools.lru_cache(maxsize=None)
def _make_sc_deg(n_acc, erows):
    rt = erows // 16
    nch = n_acc // 256
    zch = (nch + 15) // 16
    mesh = plsc.VectorSubcoreMesh(core_axis_name="c", subcore_axis_name="s")

    def body(srcd, dstp, out_o, out_i, idxb, onesb, zbuf, acc):
        cid = lax.axis_index("c")
        sid = lax.axis_index("s")

        _fill_rows(zbuf, 256, 16, 0.0)
        _fill_rows(onesb, 128, 16, 1.0)

        def zacc(j, carry):
            ch = sid + 16 * j

            @pl.when(ch < nch)
            def _():
                pltpu.sync_copy(zbuf, acc.at[pl.ds(ch * 256, 256)])

            return carry

        lax.fori_loop(0, zch, zacc, 0)
        plsc.subcore_barrier()

        def run(idxs, out):
            def chunk(g, carry):
                base = sid * rt + g * 8
                pltpu.sync_copy(idxs.at[pl.ds(base, 8)], idxb)
                for j in range(8):
                    pltpu.sync_copy(onesb, acc.at[idxb.at[j]], add=True)
                return carry

            lax.fori_loop(0, rt // 8, chunk, 0)
            plsc.subcore_barrier()

            def wb(j, carry):
                ch = sid + 16 * j

                @pl.when(ch < nch)
                def _():
                    pltpu.sync_copy(acc.at[pl.ds(ch * 256, 256)],
                                    out.at[pl.ds(ch * 256, 256)])

                return carry

            lax.fori_loop(0, zch, wb, 0)

        @pl.when(cid == 0)
        def _():
            run(srcd, out_o)

        @pl.when(cid == 1)
        def _():
            run(dstp, out_i)

    return pl.kernel(
        body,
        mesh=mesh,
        out_type=[jax.ShapeDtypeStruct((n_acc, 16), jnp.float32),
                  jax.ShapeDtypeStruct((n_acc, 16), jnp.float32)],
        scratch_types=[
            pltpu.VMEM((8, 128), jnp.int32),
            pltpu.VMEM((128, 16), jnp.float32),
            pltpu.VMEM((256, 16), jnp.float32),
            pltpu.VMEM_SHARED((n_acc, 16), jnp.float32),
        ],
        compiler_params=pltpu.CompilerParams(use_tc_tiling_on_sc=False),
    )


_BR = 512


def _prep_body(h_ref, dgo_ref, dgi_ref, tab_ref, xs0_ref, xs1_ref, nw_ref,
               ns_ref, nd_ref):
    hb = h_ref[...]
    lane = lax.broadcasted_iota(jnp.int32, hb.shape, 1)
    hm = jnp.where(lane < 26, hb, -1.0)
    mx = jnp.max(hm, axis=1, keepdims=True)
    cat = jnp.min(jnp.where(hm == mx, lane, 1000), axis=1, keepdims=True)
    w = jnp.sum(jnp.where(lane == cat, tab_ref[...], 0.0), axis=1,
                keepdims=True)
    dgo = dgo_ref[:, 0:1]
    dgi = dgi_ref[:, 0:1]
    ns = jnp.where(dgo > 0, lax.rsqrt(jnp.maximum(dgo, 1.0)), 0.0)
    nd = jnp.where(dgi > 0, lax.rsqrt(jnp.maximum(dgi, 1.0)), 0.0)
    xs = hb * ns
    xs0_ref[...] = xs[:, :16]
    xs1_ref[...] = xs[:, 16:]
    nw_ref[...] = w
    ns_ref[...] = ns
    nd_ref[...] = nd


@functools.lru_cache(maxsize=None)
def _make_tc_prep(n, n_acc):
    gb = n_acc // _BR
    rspec = lambda w: pl.BlockSpec((_BR, w), lambda i: (i, 0))
    return pl.pallas_call(
        _prep_body,
        grid=(gb,),
        in_specs=[rspec(32), rspec(16), rspec(16),
                  pl.BlockSpec((1, 32), lambda i: (0, 0))],
        out_specs=[rspec(16), rspec(16), rspec(1), rspec(1), rspec(1)],
        out_shape=[jax.ShapeDtypeStruct((n, 16), jnp.float32),
                   jax.ShapeDtypeStruct((n, 16), jnp.float32),
                   jax.ShapeDtypeStruct((n, 1), jnp.float32),
                   jax.ShapeDtypeStruct((n, 1), jnp.float32),
                   jax.ShapeDtypeStruct((n, 1), jnp.float32)],
    )


@functools.lru_cache(maxsize=None)
def _make_tc_stats(n, n_acc, dh):
    dc = 2 * dh
    gb = n_acc // _BR

    def body(a0_ref, a1_ref, nd_ref, g_ref, s_ref, m_ref):
        i = pl.program_id(0)
        A = jnp.concatenate([a0_ref[...], a1_ref[...]], axis=1) * nd_ref[...]
        A = A.astype(jnp.bfloat16).astype(jnp.float32)
        row = lax.broadcasted_iota(jnp.int32, (A.shape[0], 1), 0) + i * _BR

        @pl.when(i == 0)
        def _():
            m_ref[...] = jnp.mean(A, axis=0, keepdims=True)

        ac = jnp.where(row < n, A - m_ref[...], 0.0)
        gp = lax.dot_general(ac, ac, (((0,), (0,)), ((), ())),
                             preferred_element_type=jnp.float32, precision=lax.Precision.HIGHEST)
        sp = jnp.sum(ac, axis=0, keepdims=True)

        @pl.when(i == 0)
        def _():
            g_ref[...] = gp
            s_ref[...] = sp

        @pl.when(i != 0)
        def _():
            g_ref[...] = g_ref[...] + gp
            s_ref[...] = s_ref[...] + sp

    rspec = lambda w: pl.BlockSpec((_BR, w), lambda i: (i, 0))
    return pl.pallas_call(
        body,
        grid=(gb,),
        in_specs=[rspec(dh), rspec(dh), rspec(1)],
        out_specs=[pl.BlockSpec((dc, dc), lambda i: (0, 0)),
                   pl.BlockSpec((1, dc), lambda i: (0, 0)),
                   pl.BlockSpec((1, dc), lambda i: (0, 0))],
        out_shape=[jax.ShapeDtypeStruct((dc, dc), jnp.float32),
                   jax.ShapeDtypeStruct((1, dc), jnp.float32),
                   jax.ShapeDtypeStruct((1, dc), jnp.float32)],
    )


@functools.lru_cache(maxsize=None)
def _make_tc_apply(n, n_acc, dh, last):
    dc = 2 * dh
    gb = n_acc // _BR

    def body(*refs):
        if last:
            (a0_ref, a1_ref, nd_ref, g_ref, s_ref, m_ref, w_ref, b_ref,
             ga_ref, be_ref, fcw_ref, fcb_ref, out_ref) = refs
        else:
            (a0_ref, a1_ref, nd_ref, g_ref, s_ref, m_ref, w_ref, b_ref,
             ga_ref, be_ref, ns_ref, xs0_ref, xs1_ref) = refs
        A = jnp.concatenate([a0_ref[...], a1_ref[...]], axis=1) * nd_ref[...]
        a16 = A.astype(jnp.bfloat16)
        w16 = w_ref[...].astype(jnp.bfloat16)
        W = w16.astype(jnp.float32)
        b = b_ref[...]
        y = jnp.dot(a16, w16, preferred_element_type=jnp.float32) + b
        mean_a = m_ref[...] + s_ref[...] / n
        ec = jnp.dot(mean_a, W, preferred_element_type=jnp.float32, precision=lax.Precision.HIGHEST)
        gw = jnp.dot(g_ref[...], W, preferred_element_type=jnp.float32, precision=lax.Precision.HIGHEST)
        sw = jnp.dot(s_ref[...], W, preferred_element_type=jnp.float32, precision=lax.Precision.HIGHEST)
        var = (jnp.sum(W * gw, axis=0, keepdims=True) - sw * sw / n) / n
        mu = ec + b
        z = (y - mu) / jnp.sqrt(var + _EPS) * ga_ref[...] + be_ref[...]
        r = jnp.maximum(z, 0.0)
        if last:
            out_ref[...] = (jnp.dot(r.astype(jnp.bfloat16),
                                    fcw_ref[...].astype(jnp.bfloat16),
                                    preferred_element_type=jnp.float32)
                            + fcb_ref[...])
        else:
            xs = r * ns_ref[...]
            xs0_ref[...] = xs[:, :32]
            xs1_ref[...] = xs[:, 32:]

    rspec = lambda w: pl.BlockSpec((_BR, w), lambda i: (i, 0))
    fspec = lambda a, c: pl.BlockSpec((a, c), lambda i: (0, 0))
    in_specs = [rspec(dh), rspec(dh), rspec(1), fspec(dc, dc), fspec(1, dc),
                fspec(1, dc), fspec(dc, 64), fspec(1, 64), fspec(1, 64),
                fspec(1, 64)]
    if last:
        in_specs += [fspec(64, 64), fspec(1, 64)]
        out_specs = [rspec(64)]
        out_shape = [jax.ShapeDtypeStruct((n, 64), jnp.float32)]
    else:
        in_specs += [rspec(1)]
        out_specs = [rspec(32), rspec(32)]
        out_shape = [jax.ShapeDtypeStruct((n, 32), jnp.float32),
                     jax.ShapeDtypeStruct((n, 32), jnp.float32)]
    return pl.pallas_call(
        body, grid=(gb,), in_specs=in_specs, out_specs=out_specs,
        out_shape=out_shape,
    )


def kernel(h, edge_index, W0, b0, W1, b1, W2, b2, g0, be0, g1, be1, g2, be2,
           fcW, fcb):
    n = h.shape[0]
    e = edge_index.shape[1]
    n_acc = ((n + 1 + 511) // 512) * 512
    epad = ((e + 16383) // 16384) * 16384
    erows = epad // 128

    src = edge_index[0]
    dst = edge_index[1]
    pad0 = jnp.zeros((epad - e,), jnp.int32)
    padn = jnp.full((epad - e,), n, jnp.int32)
    srcg = jnp.concatenate([src, pad0]).reshape(erows, 128)
    srcd = jnp.concatenate([src, padn]).reshape(erows, 128)
    dstp = jnp.concatenate([dst, padn]).reshape(erows, 128)

    h32 = jnp.pad(h, ((0, 0), (0, 32 - h.shape[1])))
    tab32 = jnp.zeros((1, 32), jnp.float32).at[0, :26].set(
        jnp.array(_WTAB, dtype=jnp.float32))
    w0p = jnp.zeros((32, 64), jnp.float32).at[:26, :].set(W0)

    deg_o, deg_i = _make_sc_deg(n_acc, erows)(srcd, dstp)
    xs0, xs1, node_w, ns, nd = _make_tc_prep(n, n_acc)(h32, deg_o, deg_i,
                                                       tab32)

    layers = [(w0p, b0, g0, be0), (W1, b1, g1, be1), (W2, b2, g2, be2)]
    for li, (W, b, g, be) in enumerate(layers):
        dh = xs0.shape[1]
        a0, a1 = _make_sc_agg(n_acc, erows, dh, 8 if dh == 16 else 4)(
            xs0, xs1, srcg, dstp)
        gmat, svec, mvec = _make_tc_stats(n, n_acc, dh)(a0, a1, nd)
        common = (a0, a1, nd, gmat, svec, mvec, W, b.reshape(1, 64),
                  g.reshape(1, 64), be.reshape(1, 64))
        if li < 2:
            xs0, xs1 = _make_tc_apply(n, n_acc, dh, False)(*common, ns)
        else:
            out, = _make_tc_apply(n, n_acc, dh, True)(*common, fcW,
                                                      fcb.reshape(1, 64))
    return (out, node_w)


# speedup vs baseline: 1.0081x; 1.0081x over previous
import functools

import jax
import jax.numpy as jnp
from jax import lax
from jax.experimental import pallas as pl
from jax.experimental.pallas import tpu as pltpu
from jax.experimental.pallas import tpu_sc as plsc

_WTAB = [0.7, 0.9, 0.7, 0.9, 0.3, 0.7, 0.3, 0.9, 0.3, 0.3, 0.9, 0.7, 0.1,
         0.9, 0.5, 0.9, 0.5, 0.5, 0.1, 0.3, 0.7, 0.9, 0.9, 0.9, 0.9, 0.9]
_EPS = 1e-5


def _fill_rows(buf, nrows, ncols, value):
    vec = jnp.full((16,), value, jnp.float32)

    def row(i, carry):
        for c in range(ncols // 16):
            buf[i, pl.ds(c * 16, 16)] = vec
        return carry

    lax.fori_loop(0, nrows, row, 0)


@functools.lru_cache(maxsize=None)
def _make_sc_agg(n_acc, erows, dh, chr_):
    rt = erows // 16
    nch = n_acc // 128
    zch = (nch + 15) // 16
    mesh = plsc.VectorSubcoreMesh(core_axis_name="c", subcore_axis_name="s")

    def body(xs0, xs1, srcp, dstp, out0, out1, sidx, didx, rows, zbuf, acc, sem):
        cid = lax.axis_index("c")
        sid = lax.axis_index("s")

        _fill_rows(zbuf, 128, dh, 0.0)

        def zacc(j, carry):
            ch = sid + 16 * j

            @pl.when(ch < nch)
            def _():
                pltpu.sync_copy(zbuf, acc.at[pl.ds(ch * 128, 128)])

            return carry

        lax.fori_loop(0, zch, zacc, 0)
        plsc.subcore_barrier()

        def run(xs, out):
            def chunk(g, carry):
                base = sid * rt + g * chr_
                pltpu.sync_copy(srcp.at[pl.ds(base, chr_)], sidx)
                pltpu.sync_copy(dstp.at[pl.ds(base, chr_)], didx)
                cps = [pltpu.async_copy(xs.at[sidx.at[j]], rows.at[j], sem)
                       for j in range(chr_)]
                for cp in cps:
                    cp.wait()
                for j in range(chr_):
                    pltpu.sync_copy(rows.at[j], acc.at[didx.at[j]], add=True)
                return carry

            lax.fori_loop(0, rt // chr_, chunk, 0)
            plsc.subcore_barrier()

            def wb(j, carry):
                ch = sid + 16 * j

                @pl.when(ch < nch)
                def _():
                    pltpu.sync_copy(acc.at[pl.ds(ch * 128, 128)],
                                    out.at[pl.ds(ch * 128, 128)])

                return carry

            lax.fori_loop(0, zch, wb, 0)

        @pl.when(cid == 0)
        def _():
            run(xs0, out0)

        @pl.when(cid == 1)
        def _():
            run(xs1, out1)

    return pl.kernel(
        body,
        mesh=mesh,
        out_type=[jax.ShapeDtypeStruct((n_acc, dh), jnp.float32),
                  jax.ShapeDtypeStruct((n_acc, dh), jnp.float32)],
        scratch_types=[
            pltpu.VMEM((chr_, 128), jnp.int32),
            pltpu.VMEM((chr_, 128), jnp.int32),
            pltpu.VMEM((chr_, 128, dh), jnp.float32),
            pltpu.VMEM((128, dh), jnp.float32),
            pltpu.VMEM_SHARED((n_acc, dh), jnp.float32),
            pltpu.SemaphoreType.DMA,
        ],
        compiler_params=pltpu.CompilerParams(use_tc_tiling_on_sc=False),
    )


@functools.lru_cache(maxsize=None)
def _make_sc_deg(n_acc, erows):
    rt = erows // 16
    nch = n_acc // 256
    zch = (nch + 15) // 16
    mesh = plsc.VectorSubcoreMesh(core_axis_name="c", subcore_axis_name="s")

    def body(srcd, dstp, out_o, out_i, idxb, onesb, zbuf, acc):
        cid = lax.axis_index("c")
        sid = lax.axis_index("s")

        _fill_rows(zbuf, 256, 16, 0.0)
        _fill_rows(onesb, 128, 16, 1.0)

        def zacc(j, carry):
            ch = sid + 16 * j

            @pl.when(ch < nch)
            def _():
                pltpu.sync_copy(zbuf, acc.at[pl.ds(ch * 256, 256)])

            return carry

        lax.fori_loop(0, zch, zacc, 0)
        plsc.subcore_barrier()

        def run(idxs, out):
            def chunk(g, carry):
                base = sid * rt + g * 8
                pltpu.sync_copy(idxs.at[pl.ds(base, 8)], idxb)
                for j in range(8):
                    pltpu.sync_copy(onesb, acc.at[idxb.at[j]], add=True)
                return carry

            lax.fori_loop(0, rt // 8, chunk, 0)
            plsc.subcore_barrier()

            def wb(j, carry):
                ch = sid + 16 * j

                @pl.when(ch < nch)
                def _():
                    pltpu.sync_copy(acc.at[pl.ds(ch * 256, 256)],
                                    out.at[pl.ds(ch * 256, 256)])

                return carry

            lax.fori_loop(0, zch, wb, 0)

        @pl.when(cid == 0)
        def _():
            run(srcd, out_o)

        @pl.when(cid == 1)
        def _():
            run(dstp, out_i)

    return pl.kernel(
        body,
        mesh=mesh,
        out_type=[jax.ShapeDtypeStruct((n_acc, 16), jnp.float32),
                  jax.ShapeDtypeStruct((n_acc, 16), jnp.float32)],
        scratch_types=[
            pltpu.VMEM((8, 128), jnp.int32),
            pltpu.VMEM((128, 16), jnp.float32),
            pltpu.VMEM((256, 16), jnp.float32),
            pltpu.VMEM_SHARED((n_acc, 16), jnp.float32),
        ],
        compiler_params=pltpu.CompilerParams(use_tc_tiling_on_sc=False),
    )


_BR = 512


def _prep_body(h_ref, dgo_ref, dgi_ref, tab_ref, xs0_ref, xs1_ref, nw_ref,
               ns_ref, nd_ref):
    hb = h_ref[...]
    lane = lax.broadcasted_iota(jnp.int32, hb.shape, 1)
    hm = jnp.where(lane < 26, hb, -1.0)
    mx = jnp.max(hm, axis=1, keepdims=True)
    cat = jnp.min(jnp.where(hm == mx, lane, 1000), axis=1, keepdims=True)
    w = jnp.sum(jnp.where(lane == cat, tab_ref[...], 0.0), axis=1,
                keepdims=True)
    dgo = dgo_ref[:, 0:1]
    dgi = dgi_ref[:, 0:1]
    ns = jnp.where(dgo > 0, lax.rsqrt(jnp.maximum(dgo, 1.0)), 0.0)
    nd = jnp.where(dgi > 0, lax.rsqrt(jnp.maximum(dgi, 1.0)), 0.0)
    xs = hb * ns
    xs0_ref[...] = xs[:, :16]
    xs1_ref[...] = xs[:, 16:]
    nw_ref[...] = w
    ns_ref[...] = ns
    nd_ref[...] = nd


@functools.lru_cache(maxsize=None)
def _make_tc_prep(n, n_acc):
    gb = n_acc // _BR
    rspec = lambda w: pl.BlockSpec((_BR, w), lambda i: (i, 0))
    return pl.pallas_call(
        _prep_body,
        grid=(gb,),
        in_specs=[rspec(32), rspec(16), rspec(16),
                  pl.BlockSpec((1, 32), lambda i: (0, 0))],
        out_specs=[rspec(16), rspec(16), rspec(1), rspec(1), rspec(1)],
        out_shape=[jax.ShapeDtypeStruct((n, 16), jnp.float32),
                   jax.ShapeDtypeStruct((n, 16), jnp.float32),
                   jax.ShapeDtypeStruct((n, 1), jnp.float32),
                   jax.ShapeDtypeStruct((n, 1), jnp.float32),
                   jax.ShapeDtypeStruct((n, 1), jnp.float32)],
    )


@functools.lru_cache(maxsize=None)
def _make_tc_stats(n, n_acc, dh):
    dc = 2 * dh
    gb = n_acc // _BR

    def body(a0_ref, a1_ref, nd_ref, g_ref, s_ref, m_ref):
        i = pl.program_id(0)
        A = jnp.concatenate([a0_ref[...], a1_ref[...]], axis=1) * nd_ref[...]
        A = A.astype(jnp.bfloat16).astype(jnp.float32)
        row = lax.broadcasted_iota(jnp.int32, (A.shape[0], 1), 0) + i * _BR

        @pl.when(i == 0)
        def _():
            m_ref[...] = jnp.mean(A, axis=0, keepdims=True)

        ac = jnp.where(row < n, A - m_ref[...], 0.0)
        gp = lax.dot_general(ac, ac, (((0,), (0,)), ((), ())),
                             preferred_element_type=jnp.float32, precision=lax.Precision.HIGHEST)
        sp = jnp.sum(ac, axis=0, keepdims=True)

        @pl.when(i == 0)
        def _():
            g_ref[...] = gp
            s_ref[...] = sp

        @pl.when(i != 0)
        def _():
            g_ref[...] = g_ref[...] + gp
            s_ref[...] = s_ref[...] + sp

    rspec = lambda w: pl.BlockSpec((_BR, w), lambda i: (i, 0))
    return pl.pallas_call(
        body,
        grid=(gb,),
        in_specs=[rspec(dh), rspec(dh), rspec(1)],
        out_specs=[pl.BlockSpec((dc, dc), lambda i: (0, 0)),
                   pl.BlockSpec((1, dc), lambda i: (0, 0)),
                   pl.BlockSpec((1, dc), lambda i: (0, 0))],
        out_shape=[jax.ShapeDtypeStruct((dc, dc), jnp.float32),
                   jax.ShapeDtypeStruct((1, dc), jnp.float32),
                   jax.ShapeDtypeStruct((1, dc), jnp.float32)],
    )


@functools.lru_cache(maxsize=None)
def _make_tc_bnfin(n, dc):
    def body(g_ref, s_ref, m_ref, w_ref, b_ref, mu_ref, den_ref):
        W = w_ref[...].astype(jnp.bfloat16).astype(jnp.float32)
        mean_a = m_ref[...] + s_ref[...] / n
        ec = jnp.dot(mean_a, W, preferred_element_type=jnp.float32,
                     precision=lax.Precision.HIGHEST)
        gw = jnp.dot(g_ref[...], W, preferred_element_type=jnp.float32,
                     precision=lax.Precision.HIGHEST)
        sw = jnp.dot(s_ref[...], W, preferred_element_type=jnp.float32,
                     precision=lax.Precision.HIGHEST)
        var = (jnp.sum(W * gw, axis=0, keepdims=True) - sw * sw / n) / n
        mu_ref[...] = ec + b_ref[...]
        den_ref[...] = jnp.sqrt(var + _EPS)

    fspec = lambda a, c: pl.BlockSpec((a, c), lambda: (0, 0))
    return pl.pallas_call(
        body,
        in_specs=[fspec(dc, dc), fspec(1, dc), fspec(1, dc), fspec(dc, 64),
                  fspec(1, 64)],
        out_specs=[fspec(1, 64), fspec(1, 64)],
        out_shape=[jax.ShapeDtypeStruct((1, 64), jnp.float32),
                   jax.ShapeDtypeStruct((1, 64), jnp.float32)],
    )


@functools.lru_cache(maxsize=None)
def _make_tc_apply(n, n_acc, dh, last):
    dc = 2 * dh
    gb = n_acc // _BR

    def body(*refs):
        if last:
            (a0_ref, a1_ref, nd_ref, mu_ref, den_ref, w_ref, b_ref,
             ga_ref, be_ref, fcw_ref, fcb_ref, out_ref) = refs
        else:
            (a0_ref, a1_ref, nd_ref, mu_ref, den_ref, w_ref, b_ref,
             ga_ref, be_ref, ns_ref, xs0_ref, xs1_ref) = refs
        A = jnp.concatenate([a0_ref[...], a1_ref[...]], axis=1) * nd_ref[...]
        a16 = A.astype(jnp.bfloat16)
        w16 = w_ref[...].astype(jnp.bfloat16)
        y = jnp.dot(a16, w16, preferred_element_type=jnp.float32) + b_ref[...]
        z = (y - mu_ref[...]) / den_ref[...] * ga_ref[...] + be_ref[...]
        r = jnp.maximum(z, 0.0)
        if last:
            out_ref[...] = (jnp.dot(r.astype(jnp.bfloat16),
                                    fcw_ref[...].astype(jnp.bfloat16),
                                    preferred_element_type=jnp.float32)
                            + fcb_ref[...])
        else:
            xs = r * ns_ref[...]
            xs0_ref[...] = xs[:, :32]
            xs1_ref[...] = xs[:, 32:]

    rspec = lambda w: pl.BlockSpec((_BR, w), lambda i: (i, 0))
    fspec = lambda a, c: pl.BlockSpec((a, c), lambda i: (0, 0))
    in_specs = [rspec(dh), rspec(dh), rspec(1), fspec(1, 64), fspec(1, 64),
                fspec(dc, 64), fspec(1, 64), fspec(1, 64), fspec(1, 64)]
    if last:
        in_specs += [fspec(64, 64), fspec(1, 64)]
        out_specs = [rspec(64)]
        out_shape = [jax.ShapeDtypeStruct((n, 64), jnp.float32)]
    else:
        in_specs += [rspec(1)]
        out_specs = [rspec(32), rspec(32)]
        out_shape = [jax.ShapeDtypeStruct((n, 32), jnp.float32),
                     jax.ShapeDtypeStruct((n, 32), jnp.float32)]
    return pl.pallas_call(
        body, grid=(gb,), in_specs=in_specs, out_specs=out_specs,
        out_shape=out_shape,
    )


def kernel(h, edge_index, W0, b0, W1, b1, W2, b2, g0, be0, g1, be1, g2, be2,
           fcW, fcb):
    n = h.shape[0]
    e = edge_index.shape[1]
    n_acc = ((n + 1 + 511) // 512) * 512
    epad = ((e + 16383) // 16384) * 16384
    erows = epad // 128

    src = edge_index[0]
    dst = edge_index[1]
    pad0 = jnp.zeros((epad - e,), jnp.int32)
    padn = jnp.full((epad - e,), n, jnp.int32)
    srcg = jnp.concatenate([src, pad0]).reshape(erows, 128)
    srcd = jnp.concatenate([src, padn]).reshape(erows, 128)
    dstp = jnp.concatenate([dst, padn]).reshape(erows, 128)

    h32 = jnp.pad(h, ((0, 0), (0, 32 - h.shape[1])))
    tab32 = jnp.zeros((1, 32), jnp.float32).at[0, :26].set(
        jnp.array(_WTAB, dtype=jnp.float32))
    w0p = jnp.zeros((32, 64), jnp.float32).at[:26, :].set(W0)

    deg_o, deg_i = _make_sc_deg(n_acc, erows)(srcd, dstp)
    xs0, xs1, node_w, ns, nd = _make_tc_prep(n, n_acc)(h32, deg_o, deg_i,
                                                       tab32)

    layers = [(w0p, b0, g0, be0), (W1, b1, g1, be1), (W2, b2, g2, be2)]
    for li, (W, b, g, be) in enumerate(layers):
        dh = xs0.shape[1]
        a0, a1 = _make_sc_agg(n_acc, erows, dh, 8 if dh == 16 else 4)(
            xs0, xs1, srcg, dstp)
        gmat, svec, mvec = _make_tc_stats(n, n_acc, dh)(a0, a1, nd)
        mu, den = _make_tc_bnfin(n, 2 * dh)(gmat, svec, mvec, W,
                                            b.reshape(1, 64))
        common = (a0, a1, nd, mu, den, W, b.reshape(1, 64),
                  g.reshape(1, 64), be.reshape(1, 64))
        if li < 2:
            xs0, xs1 = _make_tc_apply(n, n_acc, dh, False)(*common, ns)
        else:
            out, = _make_tc_apply(n, n_acc, dh, True)(*common, fcW,
                                                      fcb.reshape(1, 64))
    return (out, node_w)


# speedup vs baseline: 1.1854x; 1.1759x over previous
import functools

import jax
import jax.numpy as jnp
from jax import lax
from jax.experimental import pallas as pl
from jax.experimental.pallas import tpu as pltpu
from jax.experimental.pallas import tpu_sc as plsc

_WTAB = [0.7, 0.9, 0.7, 0.9, 0.3, 0.7, 0.3, 0.9, 0.3, 0.3, 0.9, 0.7, 0.1,
         0.9, 0.5, 0.9, 0.5, 0.5, 0.1, 0.3, 0.7, 0.9, 0.9, 0.9, 0.9, 0.9]
_EPS = 1e-5


def _fill_rows(buf, nrows, ncols, value):
    vec = jnp.full((16,), value, jnp.float32)

    def row(i, carry):
        for c in range(ncols // 16):
            buf[i, pl.ds(c * 16, 16)] = vec
        return carry

    lax.fori_loop(0, nrows, row, 0)


@functools.lru_cache(maxsize=None)
def _make_sc_agg(n_acc, erows, dh, chr_):
    rt = erows // 16
    nch = n_acc // 128
    zch = (nch + 15) // 16
    mesh = plsc.VectorSubcoreMesh(core_axis_name="c", subcore_axis_name="s")

    def body(xs0, xs1, srcp, dstp, out0, out1, sidx, didx, rows, zbuf, acc, sem):
        cid = lax.axis_index("c")
        sid = lax.axis_index("s")

        _fill_rows(zbuf, 128, dh, 0.0)

        def zacc(j, carry):
            ch = sid + 16 * j

            @pl.when(ch < nch)
            def _():
                pltpu.sync_copy(zbuf, acc.at[pl.ds(ch * 128, 128)])

            return carry

        lax.fori_loop(0, zch, zacc, 0)
        plsc.subcore_barrier()

        def run(xs, out):
            def chunk(g, carry):
                base = sid * rt + g * chr_
                pltpu.sync_copy(srcp.at[pl.ds(base, chr_)], sidx)
                pltpu.sync_copy(dstp.at[pl.ds(base, chr_)], didx)
                cps = [pltpu.async_copy(xs.at[sidx.at[j]], rows.at[j], sem)
                       for j in range(chr_)]
                for cp in cps:
                    cp.wait()
                for j in range(chr_):
                    pltpu.sync_copy(rows.at[j], acc.at[didx.at[j]], add=True)
                return carry

            lax.fori_loop(0, rt // chr_, chunk, 0)
            plsc.subcore_barrier()

            def wb(j, carry):
                ch = sid + 16 * j

                @pl.when(ch < nch)
                def _():
                    pltpu.sync_copy(acc.at[pl.ds(ch * 128, 128)],
                                    out.at[pl.ds(ch * 128, 128)])

                return carry

            lax.fori_loop(0, zch, wb, 0)

        @pl.when(cid == 0)
        def _():
            run(xs0, out0)

        @pl.when(cid == 1)
        def _():
            run(xs1, out1)

    return pl.kernel(
        body,
        mesh=mesh,
        out_type=[jax.ShapeDtypeStruct((n_acc, dh), jnp.float32),
                  jax.ShapeDtypeStruct((n_acc, dh), jnp.float32)],
        scratch_types=[
            pltpu.VMEM((chr_, 128), jnp.int32),
            pltpu.VMEM((chr_, 128), jnp.int32),
            pltpu.VMEM((chr_, 128, dh), jnp.float32),
            pltpu.VMEM((128, dh), jnp.float32),
            pltpu.VMEM_SHARED((n_acc, dh), jnp.float32),
            pltpu.SemaphoreType.DMA,
        ],
        compiler_params=pltpu.CompilerParams(use_tc_tiling_on_sc=False),
    )


@functools.lru_cache(maxsize=None)
def _make_sc_deg(n_acc, erows):
    rt = erows // 16
    nch = n_acc // 256
    zch = (nch + 15) // 16
    mesh = plsc.VectorSubcoreMesh(core_axis_name="c", subcore_axis_name="s")

    def body(srcd, dstp, out_o, out_i, idxb, onesb, zbuf, acc):
        cid = lax.axis_index("c")
        sid = lax.axis_index("s")

        _fill_rows(zbuf, 256, 16, 0.0)
        _fill_rows(onesb, 128, 16, 1.0)

        def zacc(j, carry):
            ch = sid + 16 * j

            @pl.when(ch < nch)
            def _():
                pltpu.sync_copy(zbuf, acc.at[pl.ds(ch * 256, 256)])

            return carry

        lax.fori_loop(0, zch, zacc, 0)
        plsc.subcore_barrier()

        def run(idxs, out):
            def chunk(g, carry):
                base = sid * rt + g * 8
                pltpu.sync_copy(idxs.at[pl.ds(base, 8)], idxb)
                for j in range(8):
                    pltpu.sync_copy(onesb, acc.at[idxb.at[j]], add=True)
                return carry

            lax.fori_loop(0, rt // 8, chunk, 0)
            plsc.subcore_barrier()

            def wb(j, carry):
                ch = sid + 16 * j

                @pl.when(ch < nch)
                def _():
                    pltpu.sync_copy(acc.at[pl.ds(ch * 256, 256)],
                                    out.at[pl.ds(ch * 256, 256)])

                return carry

            lax.fori_loop(0, zch, wb, 0)

        @pl.when(cid == 0)
        def _():
            run(srcd, out_o)

        @pl.when(cid == 1)
        def _():
            run(dstp, out_i)

    return pl.kernel(
        body,
        mesh=mesh,
        out_type=[jax.ShapeDtypeStruct((n_acc, 16), jnp.float32),
                  jax.ShapeDtypeStruct((n_acc, 16), jnp.float32)],
        scratch_types=[
            pltpu.VMEM((8, 128), jnp.int32),
            pltpu.VMEM((128, 16), jnp.float32),
            pltpu.VMEM((256, 16), jnp.float32),
            pltpu.VMEM_SHARED((n_acc, 16), jnp.float32),
        ],
        compiler_params=pltpu.CompilerParams(use_tc_tiling_on_sc=False),
    )


_BR = 1792


def _prep_body(h_ref, dgo_ref, dgi_ref, tab_ref, xs0_ref, xs1_ref, nw_ref,
               ns_ref, nd_ref):
    hb = h_ref[...]
    lane = lax.broadcasted_iota(jnp.int32, hb.shape, 1)
    hm = jnp.where(lane < 26, hb, -1.0)
    mx = jnp.max(hm, axis=1, keepdims=True)
    cat = jnp.min(jnp.where(hm == mx, lane, 1000), axis=1, keepdims=True)
    w = jnp.sum(jnp.where(lane == cat, tab_ref[...], 0.0), axis=1,
                keepdims=True)
    dgo = dgo_ref[:, 0:1]
    dgi = dgi_ref[:, 0:1]
    ns = jnp.where(dgo > 0, lax.rsqrt(jnp.maximum(dgo, 1.0)), 0.0)
    nd = jnp.where(dgi > 0, lax.rsqrt(jnp.maximum(dgi, 1.0)), 0.0)
    xs = hb * ns
    xs0_ref[...] = xs[:, :16]
    xs1_ref[...] = xs[:, 16:]
    nw_ref[...] = w
    ns_ref[...] = ns
    nd_ref[...] = nd


@functools.lru_cache(maxsize=None)
def _make_tc_prep(n, n_acc):
    gb = n_acc // _BR
    rspec = lambda w: pl.BlockSpec((_BR, w), lambda i: (i, 0))
    return pl.pallas_call(
        _prep_body,
        grid=(gb,),
        in_specs=[rspec(32), rspec(16), rspec(16),
                  pl.BlockSpec((1, 32), lambda i: (0, 0))],
        out_specs=[rspec(16), rspec(16), rspec(1), rspec(1), rspec(1)],
        out_shape=[jax.ShapeDtypeStruct((n, 16), jnp.float32),
                   jax.ShapeDtypeStruct((n, 16), jnp.float32),
                   jax.ShapeDtypeStruct((n, 1), jnp.float32),
                   jax.ShapeDtypeStruct((n, 1), jnp.float32),
                   jax.ShapeDtypeStruct((n, 1), jnp.float32)],
    )


@functools.lru_cache(maxsize=None)
def _make_tc_stats(n, n_acc, dh):
    gb = n_acc // _BR

    def body(a0_ref, a1_ref, nd_ref, w_ref, sy_ref, sy2_ref, yc_ref):
        i = pl.program_id(0)
        A = jnp.concatenate([a0_ref[...], a1_ref[...]], axis=1) * nd_ref[...]
        yp = jnp.dot(A.astype(jnp.bfloat16), w_ref[...].astype(jnp.bfloat16),
                     preferred_element_type=jnp.float32)
        row = lax.broadcasted_iota(jnp.int32, (yp.shape[0], 1), 0) + i * _BR

        @pl.when(i == 0)
        def _():
            yc_ref[...] = jnp.mean(yp, axis=0, keepdims=True)

        d = jnp.where(row < n, yp - yc_ref[...], 0.0)
        sp = jnp.sum(d, axis=0, keepdims=True)
        qp = jnp.sum(d * d, axis=0, keepdims=True)

        @pl.when(i == 0)
        def _():
            sy_ref[...] = sp
            sy2_ref[...] = qp

        @pl.when(i != 0)
        def _():
            sy_ref[...] = sy_ref[...] + sp
            sy2_ref[...] = sy2_ref[...] + qp

    rspec = lambda w: pl.BlockSpec((_BR, w), lambda i: (i, 0))
    return pl.pallas_call(
        body,
        grid=(gb,),
        in_specs=[rspec(dh), rspec(dh), rspec(1),
                  pl.BlockSpec((2 * dh, 64), lambda i: (0, 0))],
        out_specs=[pl.BlockSpec((1, 64), lambda i: (0, 0))] * 3,
        out_shape=[jax.ShapeDtypeStruct((1, 64), jnp.float32)] * 3,
    )


@functools.lru_cache(maxsize=None)
def _make_tc_bnfin(n):
    def body(sy_ref, sy2_ref, yc_ref, b_ref, mu_ref, den_ref):
        sm = sy_ref[...] / n
        mu_ref[...] = yc_ref[...] + sm + b_ref[...]
        den_ref[...] = jnp.sqrt(sy2_ref[...] / n - sm * sm + _EPS)

    fspec = pl.BlockSpec((1, 64), lambda: (0, 0))
    return pl.pallas_call(
        body,
        in_specs=[fspec] * 4,
        out_specs=[fspec] * 2,
        out_shape=[jax.ShapeDtypeStruct((1, 64), jnp.float32)] * 2,
    )


@functools.lru_cache(maxsize=None)
def _make_tc_apply(n, n_acc, dh, last):
    dc = 2 * dh
    gb = n_acc // _BR

    def body(*refs):
        if last:
            (a0_ref, a1_ref, nd_ref, mu_ref, den_ref, w_ref, b_ref,
             ga_ref, be_ref, fcw_ref, fcb_ref, out_ref) = refs
        else:
            (a0_ref, a1_ref, nd_ref, mu_ref, den_ref, w_ref, b_ref,
             ga_ref, be_ref, ns_ref, xs0_ref, xs1_ref) = refs
        A = jnp.concatenate([a0_ref[...], a1_ref[...]], axis=1) * nd_ref[...]
        a16 = A.astype(jnp.bfloat16)
        w16 = w_ref[...].astype(jnp.bfloat16)
        y = jnp.dot(a16, w16, preferred_element_type=jnp.float32) + b_ref[...]
        z = (y - mu_ref[...]) / den_ref[...] * ga_ref[...] + be_ref[...]
        r = jnp.maximum(z, 0.0)
        if last:
            out_ref[...] = (jnp.dot(r.astype(jnp.bfloat16),
                                    fcw_ref[...].astype(jnp.bfloat16),
                                    preferred_element_type=jnp.float32)
                            + fcb_ref[...])
        else:
            xs = r * ns_ref[...]
            xs0_ref[...] = xs[:, :32]
            xs1_ref[...] = xs[:, 32:]

    rspec = lambda w: pl.BlockSpec((_BR, w), lambda i: (i, 0))
    fspec = lambda a, c: pl.BlockSpec((a, c), lambda i: (0, 0))
    in_specs = [rspec(dh), rspec(dh), rspec(1), fspec(1, 64), fspec(1, 64),
                fspec(dc, 64), fspec(1, 64), fspec(1, 64), fspec(1, 64)]
    if last:
        in_specs += [fspec(64, 64), fspec(1, 64)]
        out_specs = [rspec(64)]
        out_shape = [jax.ShapeDtypeStruct((n, 64), jnp.float32)]
    else:
        in_specs += [rspec(1)]
        out_specs = [rspec(32), rspec(32)]
        out_shape = [jax.ShapeDtypeStruct((n, 32), jnp.float32),
                     jax.ShapeDtypeStruct((n, 32), jnp.float32)]
    return pl.pallas_call(
        body, grid=(gb,), in_specs=in_specs, out_specs=out_specs,
        out_shape=out_shape,
    )


def kernel(h, edge_index, W0, b0, W1, b1, W2, b2, g0, be0, g1, be1, g2, be2,
           fcW, fcb):
    n = h.shape[0]
    e = edge_index.shape[1]
    n_acc = ((n + 1 + 511) // 512) * 512
    epad = ((e + 16383) // 16384) * 16384
    erows = epad // 128

    src = edge_index[0]
    dst = edge_index[1]
    pad0 = jnp.zeros((epad - e,), jnp.int32)
    padn = jnp.full((epad - e,), n, jnp.int32)
    srcg = jnp.concatenate([src, pad0]).reshape(erows, 128)
    srcd = jnp.concatenate([src, padn]).reshape(erows, 128)
    dstp = jnp.concatenate([dst, padn]).reshape(erows, 128)

    h32 = jnp.pad(h, ((0, 0), (0, 32 - h.shape[1])))
    tab32 = jnp.zeros((1, 32), jnp.float32).at[0, :26].set(
        jnp.array(_WTAB, dtype=jnp.float32))
    w0p = jnp.zeros((32, 64), jnp.float32).at[:26, :].set(W0)

    deg_o, deg_i = _make_sc_deg(n_acc, erows)(srcd, dstp)
    xs0, xs1, node_w, ns, nd = _make_tc_prep(n, n_acc)(h32, deg_o, deg_i,
                                                       tab32)

    layers = [(w0p, b0, g0, be0), (W1, b1, g1, be1), (W2, b2, g2, be2)]
    for li, (W, b, g, be) in enumerate(layers):
        dh = xs0.shape[1]
        a0, a1 = _make_sc_agg(n_acc, erows, dh, 8 if dh == 16 else 4)(
            xs0, xs1, srcg, dstp)
        sy, sy2, yc = _make_tc_stats(n, n_acc, dh)(a0, a1, nd, W)
        mu, den = _make_tc_bnfin(n)(sy, sy2, yc, b.reshape(1, 64))
        common = (a0, a1, nd, mu, den, W, b.reshape(1, 64),
                  g.reshape(1, 64), be.reshape(1, 64))
        if li < 2:
            xs0, xs1 = _make_tc_apply(n, n_acc, dh, False)(*common, ns)
        else:
            out, = _make_tc_apply(n, n_acc, dh, True)(*common, fcW,
                                                      fcb.reshape(1, 64))
    return (out, node_w)


# speedup vs baseline: 1.2741x; 1.0748x over previous
import functools

import jax
import jax.numpy as jnp
from jax import lax
from jax.experimental import pallas as pl
from jax.experimental.pallas import tpu as pltpu
from jax.experimental.pallas import tpu_sc as plsc

_WTAB = [0.7, 0.9, 0.7, 0.9, 0.3, 0.7, 0.3, 0.9, 0.3, 0.3, 0.9, 0.7, 0.1,
         0.9, 0.5, 0.9, 0.5, 0.5, 0.1, 0.3, 0.7, 0.9, 0.9, 0.9, 0.9, 0.9]
_EPS = 1e-5


def _fill_rows(buf, nrows, ncols, value):
    vec = jnp.full((16,), value, jnp.float32)

    def row(i, carry):
        for c in range(ncols // 16):
            buf[i, pl.ds(c * 16, 16)] = vec
        return carry

    lax.fori_loop(0, nrows, row, 0)


@functools.lru_cache(maxsize=None)
def _make_sc_agg(n_acc, erows, dh, chr_):
    rt = erows // 16
    nch = n_acc // 128
    zch = (nch + 15) // 16
    mesh = plsc.VectorSubcoreMesh(core_axis_name="c", subcore_axis_name="s")

    nchunks = rt // chr_

    def body(xs0, xs1, srcp, dstp, out0, out1, sidx2, didx2, rows2, zbuf,
             acc, sem0, sem1):
        cid = lax.axis_index("c")
        sid = lax.axis_index("s")
        sems = (sem0, sem1)

        _fill_rows(zbuf, 128, dh, 0.0)

        def zacc(j, carry):
            ch = sid + 16 * j

            @pl.when(ch < nch)
            def _():
                pltpu.sync_copy(zbuf, acc.at[pl.ds(ch * 128, 128)])

            return carry

        lax.fori_loop(0, zch, zacc, 0)
        plsc.subcore_barrier()

        def run(xs, out):
            def fire(b, cur):
                base = sid * rt + cur * chr_
                sb = sidx2.at[b]
                db = didx2.at[b]
                rb = rows2.at[b]
                pltpu.sync_copy(srcp.at[pl.ds(base, chr_)], sb)
                pltpu.sync_copy(dstp.at[pl.ds(base, chr_)], db)
                for j in range(chr_):
                    pltpu.async_copy(xs.at[sb.at[j]], rb.at[j], sems[b])

            def drain(b):
                sb = sidx2.at[b]
                db = didx2.at[b]
                rb = rows2.at[b]
                for j in range(chr_):
                    pltpu.make_async_copy(xs.at[sb.at[j]], rb.at[j],
                                          sems[b]).wait()
                for j in range(chr_):
                    pltpu.sync_copy(rb.at[j], acc.at[db.at[j]], add=True)

            fire(0, 0)

            def pair(g2, carry):
                cur = 2 * g2
                fire(1, cur + 1)
                drain(0)

                @pl.when(cur + 2 < nchunks)
                def _():
                    fire(0, cur + 2)

                drain(1)
                return carry

            lax.fori_loop(0, nchunks // 2, pair, 0)
            plsc.subcore_barrier()

            def wb(j, carry):
                ch = sid + 16 * j

                @pl.when(ch < nch)
                def _():
                    pltpu.sync_copy(acc.at[pl.ds(ch * 128, 128)],
                                    out.at[pl.ds(ch * 128, 128)])

                return carry

            lax.fori_loop(0, zch, wb, 0)

        @pl.when(cid == 0)
        def _():
            run(xs0, out0)

        @pl.when(cid == 1)
        def _():
            run(xs1, out1)

    return pl.kernel(
        body,
        mesh=mesh,
        out_type=[jax.ShapeDtypeStruct((n_acc, dh), jnp.float32),
                  jax.ShapeDtypeStruct((n_acc, dh), jnp.float32)],
        scratch_types=[
            pltpu.VMEM((2, chr_, 128), jnp.int32),
            pltpu.VMEM((2, chr_, 128), jnp.int32),
            pltpu.VMEM((2, chr_, 128, dh), jnp.float32),
            pltpu.VMEM((128, dh), jnp.float32),
            pltpu.VMEM_SHARED((n_acc, dh), jnp.float32),
            pltpu.SemaphoreType.DMA,
            pltpu.SemaphoreType.DMA,
        ],
        compiler_params=pltpu.CompilerParams(use_tc_tiling_on_sc=False),
    )


@functools.lru_cache(maxsize=None)
def _make_sc_deg(n_acc, erows):
    rt = erows // 16
    nch = n_acc // 256
    zch = (nch + 15) // 16
    mesh = plsc.VectorSubcoreMesh(core_axis_name="c", subcore_axis_name="s")

    def body(srcd, dstp, out_o, out_i, idxb, onesb, zbuf, acc):
        cid = lax.axis_index("c")
        sid = lax.axis_index("s")

        _fill_rows(zbuf, 256, 16, 0.0)
        _fill_rows(onesb, 128, 16, 1.0)

        def zacc(j, carry):
            ch = sid + 16 * j

            @pl.when(ch < nch)
            def _():
                pltpu.sync_copy(zbuf, acc.at[pl.ds(ch * 256, 256)])

            return carry

        lax.fori_loop(0, zch, zacc, 0)
        plsc.subcore_barrier()

        def run(idxs, out):
            def chunk(g, carry):
                base = sid * rt + g * 8
                pltpu.sync_copy(idxs.at[pl.ds(base, 8)], idxb)
                for j in range(8):
                    pltpu.sync_copy(onesb, acc.at[idxb.at[j]], add=True)
                return carry

            lax.fori_loop(0, rt // 8, chunk, 0)
            plsc.subcore_barrier()

            def wb(j, carry):
                ch = sid + 16 * j

                @pl.when(ch < nch)
                def _():
                    pltpu.sync_copy(acc.at[pl.ds(ch * 256, 256)],
                                    out.at[pl.ds(ch * 256, 256)])

                return carry

            lax.fori_loop(0, zch, wb, 0)

        @pl.when(cid == 0)
        def _():
            run(srcd, out_o)

        @pl.when(cid == 1)
        def _():
            run(dstp, out_i)

    return pl.kernel(
        body,
        mesh=mesh,
        out_type=[jax.ShapeDtypeStruct((n_acc, 16), jnp.float32),
                  jax.ShapeDtypeStruct((n_acc, 16), jnp.float32)],
        scratch_types=[
            pltpu.VMEM((8, 128), jnp.int32),
            pltpu.VMEM((128, 16), jnp.float32),
            pltpu.VMEM((256, 16), jnp.float32),
            pltpu.VMEM_SHARED((n_acc, 16), jnp.float32),
        ],
        compiler_params=pltpu.CompilerParams(use_tc_tiling_on_sc=False),
    )


_BR = 1792


def _prep_body(h_ref, dgo_ref, dgi_ref, tab_ref, xs0_ref, xs1_ref, nw_ref,
               ns_ref, nd_ref):
    hb = h_ref[...]
    lane = lax.broadcasted_iota(jnp.int32, hb.shape, 1)
    hm = jnp.where(lane < 26, hb, -1.0)
    mx = jnp.max(hm, axis=1, keepdims=True)
    cat = jnp.min(jnp.where(hm == mx, lane, 1000), axis=1, keepdims=True)
    w = jnp.sum(jnp.where(lane == cat, tab_ref[...], 0.0), axis=1,
                keepdims=True)
    dgo = dgo_ref[:, 0:1]
    dgi = dgi_ref[:, 0:1]
    ns = jnp.where(dgo > 0, lax.rsqrt(jnp.maximum(dgo, 1.0)), 0.0)
    nd = jnp.where(dgi > 0, lax.rsqrt(jnp.maximum(dgi, 1.0)), 0.0)
    xs = hb * ns
    xs0_ref[...] = xs[:, :16]
    xs1_ref[...] = xs[:, 16:]
    nw_ref[...] = w
    ns_ref[...] = ns
    nd_ref[...] = nd


@functools.lru_cache(maxsize=None)
def _make_tc_prep(n, n_acc):
    gb = n_acc // _BR
    rspec = lambda w: pl.BlockSpec((_BR, w), lambda i: (i, 0))
    return pl.pallas_call(
        _prep_body,
        grid=(gb,),
        in_specs=[rspec(32), rspec(16), rspec(16),
                  pl.BlockSpec((1, 32), lambda i: (0, 0))],
        out_specs=[rspec(16), rspec(16), rspec(1), rspec(1), rspec(1)],
        out_shape=[jax.ShapeDtypeStruct((n, 16), jnp.float32),
                   jax.ShapeDtypeStruct((n, 16), jnp.float32),
                   jax.ShapeDtypeStruct((n, 1), jnp.float32),
                   jax.ShapeDtypeStruct((n, 1), jnp.float32),
                   jax.ShapeDtypeStruct((n, 1), jnp.float32)],
    )


@functools.lru_cache(maxsize=None)
def _make_tc_stats(n, n_acc, dh):
    gb = n_acc // _BR

    def body(a0_ref, a1_ref, nd_ref, w_ref, sy_ref, sy2_ref, yc_ref):
        i = pl.program_id(0)
        A = jnp.concatenate([a0_ref[...], a1_ref[...]], axis=1) * nd_ref[...]
        yp = jnp.dot(A.astype(jnp.bfloat16), w_ref[...].astype(jnp.bfloat16),
                     preferred_element_type=jnp.float32)
        row = lax.broadcasted_iota(jnp.int32, (yp.shape[0], 1), 0) + i * _BR

        @pl.when(i == 0)
        def _():
            yc_ref[...] = jnp.mean(yp, axis=0, keepdims=True)

        d = jnp.where(row < n, yp - yc_ref[...], 0.0)
        sp = jnp.sum(d, axis=0, keepdims=True)
        qp = jnp.sum(d * d, axis=0, keepdims=True)

        @pl.when(i == 0)
        def _():
            sy_ref[...] = sp
            sy2_ref[...] = qp

        @pl.when(i != 0)
        def _():
            sy_ref[...] = sy_ref[...] + sp
            sy2_ref[...] = sy2_ref[...] + qp

    rspec = lambda w: pl.BlockSpec((_BR, w), lambda i: (i, 0))
    return pl.pallas_call(
        body,
        grid=(gb,),
        in_specs=[rspec(dh), rspec(dh), rspec(1),
                  pl.BlockSpec((2 * dh, 64), lambda i: (0, 0))],
        out_specs=[pl.BlockSpec((1, 64), lambda i: (0, 0))] * 3,
        out_shape=[jax.ShapeDtypeStruct((1, 64), jnp.float32)] * 3,
    )


@functools.lru_cache(maxsize=None)
def _make_tc_bnfin(n):
    def body(sy_ref, sy2_ref, yc_ref, b_ref, mu_ref, den_ref):
        sm = sy_ref[...] / n
        mu_ref[...] = yc_ref[...] + sm + b_ref[...]
        den_ref[...] = jnp.sqrt(sy2_ref[...] / n - sm * sm + _EPS)

    fspec = pl.BlockSpec((1, 64), lambda: (0, 0))
    return pl.pallas_call(
        body,
        in_specs=[fspec] * 4,
        out_specs=[fspec] * 2,
        out_shape=[jax.ShapeDtypeStruct((1, 64), jnp.float32)] * 2,
    )


@functools.lru_cache(maxsize=None)
def _make_tc_apply(n, n_acc, dh, last):
    dc = 2 * dh
    gb = n_acc // _BR

    def body(*refs):
        if last:
            (a0_ref, a1_ref, nd_ref, mu_ref, den_ref, w_ref, b_ref,
             ga_ref, be_ref, fcw_ref, fcb_ref, out_ref) = refs
        else:
            (a0_ref, a1_ref, nd_ref, mu_ref, den_ref, w_ref, b_ref,
             ga_ref, be_ref, ns_ref, xs0_ref, xs1_ref) = refs
        A = jnp.concatenate([a0_ref[...], a1_ref[...]], axis=1) * nd_ref[...]
        a16 = A.astype(jnp.bfloat16)
        w16 = w_ref[...].astype(jnp.bfloat16)
        y = jnp.dot(a16, w16, preferred_element_type=jnp.float32) + b_ref[...]
        z = (y - mu_ref[...]) / den_ref[...] * ga_ref[...] + be_ref[...]
        r = jnp.maximum(z, 0.0)
        if last:
            out_ref[...] = (jnp.dot(r.astype(jnp.bfloat16),
                                    fcw_ref[...].astype(jnp.bfloat16),
                                    preferred_element_type=jnp.float32)
                            + fcb_ref[...])
        else:
            xs = r * ns_ref[...]
            xs0_ref[...] = xs[:, :32]
            xs1_ref[...] = xs[:, 32:]

    rspec = lambda w: pl.BlockSpec((_BR, w), lambda i: (i, 0))
    fspec = lambda a, c: pl.BlockSpec((a, c), lambda i: (0, 0))
    in_specs = [rspec(dh), rspec(dh), rspec(1), fspec(1, 64), fspec(1, 64),
                fspec(dc, 64), fspec(1, 64), fspec(1, 64), fspec(1, 64)]
    if last:
        in_specs += [fspec(64, 64), fspec(1, 64)]
        out_specs = [rspec(64)]
        out_shape = [jax.ShapeDtypeStruct((n, 64), jnp.float32)]
    else:
        in_specs += [rspec(1)]
        out_specs = [rspec(32), rspec(32)]
        out_shape = [jax.ShapeDtypeStruct((n, 32), jnp.float32),
                     jax.ShapeDtypeStruct((n, 32), jnp.float32)]
    return pl.pallas_call(
        body, grid=(gb,), in_specs=in_specs, out_specs=out_specs,
        out_shape=out_shape,
    )


def kernel(h, edge_index, W0, b0, W1, b1, W2, b2, g0, be0, g1, be1, g2, be2,
           fcW, fcb):
    n = h.shape[0]
    e = edge_index.shape[1]
    n_acc = ((n + 1 + 511) // 512) * 512
    epad = ((e + 16383) // 16384) * 16384
    erows = epad // 128

    src = edge_index[0]
    dst = edge_index[1]
    pad0 = jnp.zeros((epad - e,), jnp.int32)
    padn = jnp.full((epad - e,), n, jnp.int32)
    srcg = jnp.concatenate([src, pad0]).reshape(erows, 128)
    srcd = jnp.concatenate([src, padn]).reshape(erows, 128)
    dstp = jnp.concatenate([dst, padn]).reshape(erows, 128)

    h32 = jnp.pad(h, ((0, 0), (0, 32 - h.shape[1])))
    tab32 = jnp.zeros((1, 32), jnp.float32).at[0, :26].set(
        jnp.array(_WTAB, dtype=jnp.float32))
    w0p = jnp.zeros((32, 64), jnp.float32).at[:26, :].set(W0)

    deg_o, deg_i = _make_sc_deg(n_acc, erows)(srcd, dstp)
    xs0, xs1, node_w, ns, nd = _make_tc_prep(n, n_acc)(h32, deg_o, deg_i,
                                                       tab32)

    layers = [(w0p, b0, g0, be0), (W1, b1, g1, be1), (W2, b2, g2, be2)]
    for li, (W, b, g, be) in enumerate(layers):
        dh = xs0.shape[1]
        a0, a1 = _make_sc_agg(n_acc, erows, dh, 4 if dh == 16 else 2)(
            xs0, xs1, srcg, dstp)
        sy, sy2, yc = _make_tc_stats(n, n_acc, dh)(a0, a1, nd, W)
        mu, den = _make_tc_bnfin(n)(sy, sy2, yc, b.reshape(1, 64))
        common = (a0, a1, nd, mu, den, W, b.reshape(1, 64),
                  g.reshape(1, 64), be.reshape(1, 64))
        if li < 2:
            xs0, xs1 = _make_tc_apply(n, n_acc, dh, False)(*common, ns)
        else:
            out, = _make_tc_apply(n, n_acc, dh, True)(*common, fcW,
                                                      fcb.reshape(1, 64))
    return (out, node_w)


# speedup vs baseline: 1.2909x; 1.0132x over previous
import functools

import jax
import jax.numpy as jnp
from jax import lax
from jax.experimental import pallas as pl
from jax.experimental.pallas import tpu as pltpu
from jax.experimental.pallas import tpu_sc as plsc

_WTAB = [0.7, 0.9, 0.7, 0.9, 0.3, 0.7, 0.3, 0.9, 0.3, 0.3, 0.9, 0.7, 0.1,
         0.9, 0.5, 0.9, 0.5, 0.5, 0.1, 0.3, 0.7, 0.9, 0.9, 0.9, 0.9, 0.9]
_EPS = 1e-5


def _fill_rows(buf, nrows, ncols, value):
    vec = jnp.full((16,), value, jnp.float32)

    def row(i, carry):
        for c in range(ncols // 16):
            buf[i, pl.ds(c * 16, 16)] = vec
        return carry

    lax.fori_loop(0, nrows, row, 0)


@functools.lru_cache(maxsize=None)
def _make_sc_agg(n_acc, erows, dh, chr_):
    rt = erows // 16
    nch = n_acc // 128
    zch = (nch + 15) // 16
    mesh = plsc.VectorSubcoreMesh(core_axis_name="c", subcore_axis_name="s")

    nchunks = rt // chr_

    def body(xs0, xs1, ei3, out0, out1, sidx2, didx2, rows2, zbuf,
             acc, sem0, sem1):
        cid = lax.axis_index("c")
        sid = lax.axis_index("s")
        sems = (sem0, sem1)

        _fill_rows(zbuf, 128, dh, 0.0)

        def zacc(j, carry):
            ch = sid + 16 * j

            @pl.when(ch < nch)
            def _():
                pltpu.sync_copy(zbuf, acc.at[pl.ds(ch * 128, 128)])

            return carry

        lax.fori_loop(0, zch, zacc, 0)
        plsc.subcore_barrier()

        def run(xs, out):
            def fire(b, cur):
                base = sid * rt + cur * chr_
                sb = sidx2.at[b]
                db = didx2.at[b]
                rb = rows2.at[b]
                pltpu.sync_copy(ei3.at[0, pl.ds(base, chr_)], sb)
                pltpu.sync_copy(ei3.at[1, pl.ds(base, chr_)], db)
                for j in range(chr_):
                    pltpu.async_copy(xs.at[sb.at[j]], rb.at[j], sems[b])

            def drain(b):
                sb = sidx2.at[b]
                db = didx2.at[b]
                rb = rows2.at[b]
                for j in range(chr_):
                    pltpu.make_async_copy(xs.at[sb.at[j]], rb.at[j],
                                          sems[b]).wait()
                for j in range(chr_):
                    pltpu.sync_copy(rb.at[j], acc.at[db.at[j]], add=True)

            fire(0, 0)

            def pair(g2, carry):
                cur = 2 * g2
                fire(1, cur + 1)
                drain(0)

                @pl.when(cur + 2 < nchunks)
                def _():
                    fire(0, cur + 2)

                drain(1)
                return carry

            lax.fori_loop(0, nchunks // 2, pair, 0)
            plsc.subcore_barrier()

            def wb(j, carry):
                ch = sid + 16 * j

                @pl.when(ch < nch)
                def _():
                    pltpu.sync_copy(acc.at[pl.ds(ch * 128, 128)],
                                    out.at[pl.ds(ch * 128, 128)])

                return carry

            lax.fori_loop(0, zch, wb, 0)

        @pl.when(cid == 0)
        def _():
            run(xs0, out0)

        @pl.when(cid == 1)
        def _():
            run(xs1, out1)

    return pl.kernel(
        body,
        mesh=mesh,
        out_type=[jax.ShapeDtypeStruct((n_acc, dh), jnp.float32),
                  jax.ShapeDtypeStruct((n_acc, dh), jnp.float32)],
        scratch_types=[
            pltpu.VMEM((2, chr_, 128), jnp.int32),
            pltpu.VMEM((2, chr_, 128), jnp.int32),
            pltpu.VMEM((2, chr_, 128, dh), jnp.float32),
            pltpu.VMEM((128, dh), jnp.float32),
            pltpu.VMEM_SHARED((n_acc, dh), jnp.float32),
            pltpu.SemaphoreType.DMA,
            pltpu.SemaphoreType.DMA,
        ],
        compiler_params=pltpu.CompilerParams(use_tc_tiling_on_sc=False),
    )


@functools.lru_cache(maxsize=None)
def _make_sc_deg(n_acc, erows):
    rt = erows // 16
    nch = n_acc // 256
    zch = (nch + 15) // 16
    mesh = plsc.VectorSubcoreMesh(core_axis_name="c", subcore_axis_name="s")

    def body(ei3, out_o, out_i, idxb, onesb, zbuf, acc):
        cid = lax.axis_index("c")
        sid = lax.axis_index("s")

        _fill_rows(zbuf, 256, 16, 0.0)
        _fill_rows(onesb, 128, 16, 1.0)

        def zacc(j, carry):
            ch = sid + 16 * j

            @pl.when(ch < nch)
            def _():
                pltpu.sync_copy(zbuf, acc.at[pl.ds(ch * 256, 256)])

            return carry

        lax.fori_loop(0, zch, zacc, 0)
        plsc.subcore_barrier()

        def run(which, out):
            def chunk(g, carry):
                base = sid * rt + g * 8
                pltpu.sync_copy(ei3.at[which, pl.ds(base, 8)], idxb)
                for j in range(8):
                    pltpu.sync_copy(onesb, acc.at[idxb.at[j]], add=True)
                return carry

            lax.fori_loop(0, rt // 8, chunk, 0)
            plsc.subcore_barrier()

            def wb(j, carry):
                ch = sid + 16 * j

                @pl.when(ch < nch)
                def _():
                    pltpu.sync_copy(acc.at[pl.ds(ch * 256, 256)],
                                    out.at[pl.ds(ch * 256, 256)])

                return carry

            lax.fori_loop(0, zch, wb, 0)

        @pl.when(cid == 0)
        def _():
            run(0, out_o)

        @pl.when(cid == 1)
        def _():
            run(1, out_i)

    return pl.kernel(
        body,
        mesh=mesh,
        out_type=[jax.ShapeDtypeStruct((n_acc, 16), jnp.float32),
                  jax.ShapeDtypeStruct((n_acc, 16), jnp.float32)],
        scratch_types=[
            pltpu.VMEM((8, 128), jnp.int32),
            pltpu.VMEM((128, 16), jnp.float32),
            pltpu.VMEM((256, 16), jnp.float32),
            pltpu.VMEM_SHARED((n_acc, 16), jnp.float32),
        ],
        compiler_params=pltpu.CompilerParams(use_tc_tiling_on_sc=False),
    )


_BR = 1792


def _prep_body(h_ref, dgo_ref, dgi_ref, tab_ref, xs0_ref, xs1_ref, nw_ref,
               ns_ref, nd_ref):
    hb = h_ref[...]
    lane = lax.broadcasted_iota(jnp.int32, hb.shape, 1)
    hm = jnp.where(lane < 26, hb, -1.0)
    mx = jnp.max(hm, axis=1, keepdims=True)
    cat = jnp.min(jnp.where(hm == mx, lane, 1000), axis=1, keepdims=True)
    w = jnp.sum(jnp.where(lane == cat, tab_ref[...], 0.0), axis=1,
                keepdims=True)
    dgo = dgo_ref[:, 0:1]
    dgi = dgi_ref[:, 0:1]
    ns = jnp.where(dgo > 0, lax.rsqrt(jnp.maximum(dgo, 1.0)), 0.0)
    nd = jnp.where(dgi > 0, lax.rsqrt(jnp.maximum(dgi, 1.0)), 0.0)
    xs = hb * ns
    xs0_ref[...] = xs[:, :16]
    xs1_ref[...] = xs[:, 16:]
    nw_ref[...] = w
    ns_ref[...] = ns
    nd_ref[...] = nd


@functools.lru_cache(maxsize=None)
def _make_tc_prep(n, n_acc):
    gb = n_acc // _BR
    rspec = lambda w: pl.BlockSpec((_BR, w), lambda i: (i, 0))
    return pl.pallas_call(
        _prep_body,
        grid=(gb,),
        in_specs=[rspec(32), rspec(16), rspec(16),
                  pl.BlockSpec((1, 32), lambda i: (0, 0))],
        out_specs=[rspec(16), rspec(16), rspec(1), rspec(1), rspec(1)],
        out_shape=[jax.ShapeDtypeStruct((n_acc, 16), jnp.float32),
                   jax.ShapeDtypeStruct((n_acc, 16), jnp.float32),
                   jax.ShapeDtypeStruct((n, 1), jnp.float32),
                   jax.ShapeDtypeStruct((n, 1), jnp.float32),
                   jax.ShapeDtypeStruct((n, 1), jnp.float32)],
    )


@functools.lru_cache(maxsize=None)
def _make_tc_stats(n, n_acc, dh):
    gb = n_acc // _BR

    def body(a0_ref, a1_ref, nd_ref, w_ref, sy_ref, sy2_ref, yc_ref):
        i = pl.program_id(0)
        A = jnp.concatenate([a0_ref[...], a1_ref[...]], axis=1) * nd_ref[...]
        yp = jnp.dot(A.astype(jnp.bfloat16), w_ref[...].astype(jnp.bfloat16),
                     preferred_element_type=jnp.float32)
        row = lax.broadcasted_iota(jnp.int32, (yp.shape[0], 1), 0) + i * _BR

        @pl.when(i == 0)
        def _():
            yc_ref[...] = jnp.mean(yp, axis=0, keepdims=True)

        d = jnp.where(row < n, yp - yc_ref[...], 0.0)
        sp = jnp.sum(d, axis=0, keepdims=True)
        qp = jnp.sum(d * d, axis=0, keepdims=True)

        @pl.when(i == 0)
        def _():
            sy_ref[...] = sp
            sy2_ref[...] = qp

        @pl.when(i != 0)
        def _():
            sy_ref[...] = sy_ref[...] + sp
            sy2_ref[...] = sy2_ref[...] + qp

    rspec = lambda w: pl.BlockSpec((_BR, w), lambda i: (i, 0))
    return pl.pallas_call(
        body,
        grid=(gb,),
        in_specs=[rspec(dh), rspec(dh), rspec(1),
                  pl.BlockSpec((2 * dh, 64), lambda i: (0, 0))],
        out_specs=[pl.BlockSpec((1, 64), lambda i: (0, 0))] * 3,
        out_shape=[jax.ShapeDtypeStruct((1, 64), jnp.float32)] * 3,
    )


@functools.lru_cache(maxsize=None)
def _make_tc_bnfin(n):
    def body(sy_ref, sy2_ref, yc_ref, b_ref, mu_ref, den_ref):
        sm = sy_ref[...] / n
        mu_ref[...] = yc_ref[...] + sm + b_ref[...]
        den_ref[...] = jnp.sqrt(sy2_ref[...] / n - sm * sm + _EPS)

    fspec = pl.BlockSpec((1, 64), lambda: (0, 0))
    return pl.pallas_call(
        body,
        in_specs=[fspec] * 4,
        out_specs=[fspec] * 2,
        out_shape=[jax.ShapeDtypeStruct((1, 64), jnp.float32)] * 2,
    )


@functools.lru_cache(maxsize=None)
def _make_tc_apply(n, n_acc, dh, last):
    dc = 2 * dh
    gb = n_acc // _BR

    def body(*refs):
        if last:
            (a0_ref, a1_ref, nd_ref, mu_ref, den_ref, w_ref, b_ref,
             ga_ref, be_ref, fcw_ref, fcb_ref, out_ref) = refs
        else:
            (a0_ref, a1_ref, nd_ref, mu_ref, den_ref, w_ref, b_ref,
             ga_ref, be_ref, ns_ref, xs0_ref, xs1_ref) = refs
        A = jnp.concatenate([a0_ref[...], a1_ref[...]], axis=1) * nd_ref[...]
        a16 = A.astype(jnp.bfloat16)
        w16 = w_ref[...].astype(jnp.bfloat16)
        y = jnp.dot(a16, w16, preferred_element_type=jnp.float32) + b_ref[...]
        z = (y - mu_ref[...]) / den_ref[...] * ga_ref[...] + be_ref[...]
        r = jnp.maximum(z, 0.0)
        if last:
            out_ref[...] = (jnp.dot(r.astype(jnp.bfloat16),
                                    fcw_ref[...].astype(jnp.bfloat16),
                                    preferred_element_type=jnp.float32)
                            + fcb_ref[...])
        else:
            xs = r * ns_ref[...]
            xs0_ref[...] = xs[:, :32]
            xs1_ref[...] = xs[:, 32:]

    rspec = lambda w: pl.BlockSpec((_BR, w), lambda i: (i, 0))
    fspec = lambda a, c: pl.BlockSpec((a, c), lambda i: (0, 0))
    in_specs = [rspec(dh), rspec(dh), rspec(1), fspec(1, 64), fspec(1, 64),
                fspec(dc, 64), fspec(1, 64), fspec(1, 64), fspec(1, 64)]
    if last:
        in_specs += [fspec(64, 64), fspec(1, 64)]
        out_specs = [rspec(64)]
        out_shape = [jax.ShapeDtypeStruct((n, 64), jnp.float32)]
    else:
        in_specs += [rspec(1)]
        out_specs = [rspec(32), rspec(32)]
        out_shape = [jax.ShapeDtypeStruct((n_acc, 32), jnp.float32),
                     jax.ShapeDtypeStruct((n_acc, 32), jnp.float32)]
    return pl.pallas_call(
        body, grid=(gb,), in_specs=in_specs, out_specs=out_specs,
        out_shape=out_shape,
    )


def kernel(h, edge_index, W0, b0, W1, b1, W2, b2, g0, be0, g1, be1, g2, be2,
           fcW, fcb):
    n = h.shape[0]
    e = edge_index.shape[1]
    n_acc = ((n + 1 + 511) // 512) * 512
    epad = ((e + 16383) // 16384) * 16384
    erows = epad // 128

    ei3 = jnp.pad(edge_index, ((0, 0), (0, epad - e)),
                  constant_values=n).reshape(2, erows, 128)

    h32 = jnp.pad(h, ((0, 0), (0, 32 - h.shape[1])))
    tab32 = jnp.zeros((1, 32), jnp.float32).at[0, :26].set(
        jnp.array(_WTAB, dtype=jnp.float32))
    w0p = jnp.zeros((32, 64), jnp.float32).at[:26, :].set(W0)

    deg_o, deg_i = _make_sc_deg(n_acc, erows)(ei3)
    xs0, xs1, node_w, ns, nd = _make_tc_prep(n, n_acc)(h32, deg_o, deg_i,
                                                       tab32)

    layers = [(w0p, b0, g0, be0), (W1, b1, g1, be1), (W2, b2, g2, be2)]
    for li, (W, b, g, be) in enumerate(layers):
        dh = xs0.shape[1]
        a0, a1 = _make_sc_agg(n_acc, erows, dh, 4 if dh == 16 else 2)(
            xs0, xs1, ei3)
        sy, sy2, yc = _make_tc_stats(n, n_acc, dh)(a0, a1, nd, W)
        mu, den = _make_tc_bnfin(n)(sy, sy2, yc, b.reshape(1, 64))
        common = (a0, a1, nd, mu, den, W, b.reshape(1, 64),
                  g.reshape(1, 64), be.reshape(1, 64))
        if li < 2:
            xs0, xs1 = _make_tc_apply(n, n_acc, dh, False)(*common, ns)
        else:
            out, = _make_tc_apply(n, n_acc, dh, True)(*common, fcW,
                                                      fcb.reshape(1, 64))
    return (out, node_w)
